# baseline (speedup 1.0000x reference)
import functools

import jax
import jax.numpy as jnp
from jax import lax
from jax.experimental import pallas as pl
from jax.experimental.pallas import tpu as pltpu
from jax.experimental.pallas import tpu_sc as plsc

B = 16384
D = 512
NC = 2
NS = 16
NW = NC * NS
RPW = B // NW
R = 32
NCHUNK = RPW // R
L = 16
DV = D // L
NG = R // L

_mesh = plsc.VectorSubcoreMesh(core_axis_name="c", subcore_axis_name="s")


@functools.partial(
    pl.kernel,
    out_type=jax.ShapeDtypeStruct((NW, L), jnp.float32),
    mesh=_mesh,
    compiler_params=pltpu.CompilerParams(needs_layout_passes=False),
    scratch_types=[
        pltpu.VMEM((R,), jnp.int32),
        pltpu.VMEM((R, D), jnp.float32),
        pltpu.VMEM((R, D), jnp.float32),
        pltpu.VMEM((L * L,), jnp.float32),
        pltpu.VMEM((L,), jnp.float32),
        pltpu.SemaphoreType.DMA,
        pltpu.SemaphoreType.DMA,
    ],
)
def _center_loss_sc(x_hbm, labels_hbm, centers_hbm, out_hbm,
                    idx_v, x_buf, c_buf, trans_v, res_v, sem_x, sem_c):
    cid = lax.axis_index("c")
    sid = lax.axis_index("s")
    wid = cid * NS + sid
    base = wid * RPW
    iota = lax.iota(jnp.int32, L)

    def chunk_body(g, totv):
        row0 = base + g * R
        cp_x = pltpu.async_copy(x_hbm.at[pl.ds(row0, R)], x_buf, sem_x)
        pltpu.sync_copy(labels_hbm.at[pl.ds(row0, R)], idx_v)
        cp_c = pltpu.async_copy(centers_hbm.at[idx_v], c_buf, sem_c)
        cp_x.wait()
        cp_c.wait()

        for h in range(NG):
            def row_body(r16, _):
                r = h * L + r16
                accs = [jnp.zeros((L,), jnp.float32) for _ in range(4)]
                for j in range(DV):
                    d = x_buf[r, pl.ds(j * L, L)] - c_buf[r, pl.ds(j * L, L)]
                    accs[j % 4] = accs[j % 4] + d * d
                trans_v[pl.ds(r16 * L, L)] = (accs[0] + accs[1]) + (accs[2] + accs[3])
                return 0

            lax.fori_loop(0, L, row_body, 0)

            tsum = plsc.load_gather(trans_v, [iota * L])
            for k in range(1, L):
                tsum = tsum + plsc.load_gather(trans_v, [iota * L + k])
            tclip = jnp.minimum(jnp.maximum(tsum, 1e-12), 1e12)
            totv = totv + tclip
        return totv

    totv = lax.fori_loop(0, NCHUNK, chunk_body, jnp.zeros((L,), jnp.float32))

    res_v[...] = totv
    pltpu.sync_copy(res_v, out_hbm.at[wid])


def _reduce_tc_body(p_ref, o_ref):
    o_ref[0, 0] = jnp.sum(p_ref[...]) * jnp.float32(1.0 / B)


_reduce_tc = pl.pallas_call(
    _reduce_tc_body,
    out_shape=jax.ShapeDtypeStruct((1, 1), jnp.float32),
    in_specs=[pl.BlockSpec(memory_space=pltpu.VMEM)],
    out_specs=pl.BlockSpec(memory_space=pltpu.SMEM),
)


def kernel(x, labels, centers):
    partials = _center_loss_sc(x, labels.astype(jnp.int32), centers)
    return _reduce_tc(partials)[0, 0]


# speedup vs baseline: 1.3204x; 1.3204x over previous
import functools

import jax
import jax.numpy as jnp
from jax import lax
from jax.experimental import pallas as pl
from jax.experimental.pallas import tpu as pltpu
from jax.experimental.pallas import tpu_sc as plsc

B = 16384
D = 512
NC = 2
NS = 16
NW = NC * NS
RPW = B // NW
R = 32
NCHUNK = RPW // R
L = 16
DV = D // L
NG = R // L

_mesh = plsc.VectorSubcoreMesh(core_axis_name="c", subcore_axis_name="s")


@functools.partial(
    pl.kernel,
    out_type=jax.ShapeDtypeStruct((NW, L), jnp.float32),
    mesh=_mesh,
    compiler_params=pltpu.CompilerParams(needs_layout_passes=False),
    scratch_types=[
        pltpu.VMEM((R,), jnp.int32),
        pltpu.VMEM((R,), jnp.int32),
        pltpu.VMEM((R, D), jnp.float32),
        pltpu.VMEM((R, D), jnp.float32),
        pltpu.VMEM((R, D), jnp.float32),
        pltpu.VMEM((R, D), jnp.float32),
        pltpu.VMEM((L * L,), jnp.float32),
        pltpu.VMEM((L,), jnp.float32),
        pltpu.SemaphoreType.DMA,
        pltpu.SemaphoreType.DMA,
        pltpu.SemaphoreType.DMA,
        pltpu.SemaphoreType.DMA,
    ],
)
def _center_loss_sc(x_hbm, labels_hbm, centers_hbm, out_hbm,
                    idx0, idx1, xb0, xb1, cb0, cb1, trans_v, res_v,
                    sem_x0, sem_x1, sem_c0, sem_c1):
    cid = lax.axis_index("c")
    sid = lax.axis_index("s")
    wid = cid * NS + sid
    base = wid * RPW
    iota = lax.iota(jnp.int32, L)

    idxs = (idx0, idx1)
    xbs = (xb0, xb1)
    cbs = (cb0, cb1)
    sxs = (sem_x0, sem_x1)
    scs = (sem_c0, sem_c1)

    def issue(g, b):
        row0 = base + g * R
        pltpu.async_copy(x_hbm.at[pl.ds(row0, R)], xbs[b], sxs[b])
        pltpu.sync_copy(labels_hbm.at[pl.ds(row0, R)], idxs[b])
        pltpu.async_copy(centers_hbm.at[idxs[b]], cbs[b], scs[b])

    def wait(g, b):
        row0 = base + g * R
        pltpu.make_async_copy(x_hbm.at[pl.ds(row0, R)], xbs[b], sxs[b]).wait()
        pltpu.make_async_copy(centers_hbm.at[idxs[b]], cbs[b], scs[b]).wait()

    def compute(b, totv):
        x_buf = xbs[b]
        c_buf = cbs[b]
        for h in range(NG):
            def row_body(r16, _):
                r = h * L + r16
                accs = [jnp.zeros((L,), jnp.float32) for _ in range(4)]
                for j in range(DV):
                    d = x_buf[r, pl.ds(j * L, L)] - c_buf[r, pl.ds(j * L, L)]
                    accs[j % 4] = accs[j % 4] + d * d
                trans_v[pl.ds(r16 * L, L)] = (accs[0] + accs[1]) + (accs[2] + accs[3])
                return 0

            lax.fori_loop(0, L, row_body, 0)

            tsum = plsc.load_gather(trans_v, [iota * L])
            for k in range(1, L):
                tsum = tsum + plsc.load_gather(trans_v, [iota * L + k])
            tclip = jnp.minimum(jnp.maximum(tsum, 1e-12), 1e12)
            totv = totv + tclip
        return totv

    issue(0, 0)

    def pair_body(t, totv):
        g0 = t * 2
        for b in range(2):
            gg = g0 + b

            @pl.when(gg + 1 < NCHUNK)
            def _():
                issue(gg + 1, 1 - b)

            wait(gg, b)
            totv = compute(b, totv)
        return totv

    totv = lax.fori_loop(0, NCHUNK // 2, pair_body, jnp.zeros((L,), jnp.float32))

    res_v[...] = totv
    pltpu.sync_copy(res_v, out_hbm.at[wid])


def _reduce_tc_body(p_ref, o_ref):
    o_ref[0, 0] = jnp.sum(p_ref[...]) * jnp.float32(1.0 / B)


_reduce_tc = pl.pallas_call(
    _reduce_tc_body,
    out_shape=jax.ShapeDtypeStruct((1, 1), jnp.float32),
    in_specs=[pl.BlockSpec(memory_space=pltpu.VMEM)],
    out_specs=pl.BlockSpec(memory_space=pltpu.SMEM),
)


def kernel(x, labels, centers):
    partials = _center_loss_sc(x, labels.astype(jnp.int32), centers)
    return _reduce_tc(partials)[0, 0]


# speedup vs baseline: 1.3653x; 1.0340x over previous
import functools

import jax
import jax.numpy as jnp
from jax import lax
from jax.experimental import pallas as pl
from jax.experimental.pallas import tpu as pltpu
from jax.experimental.pallas import tpu_sc as plsc

B = 16384
D = 512
NC = 2
NS = 16
NW = NC * NS
RPW = B // NW
R = 32
NCHUNK = RPW // R
L = 16
DV = D // L
NG = R // L

_mesh = plsc.VectorSubcoreMesh(core_axis_name="c", subcore_axis_name="s")


@functools.partial(
    pl.kernel,
    out_type=jax.ShapeDtypeStruct((NW, L), jnp.float32),
    mesh=_mesh,
    compiler_params=pltpu.CompilerParams(needs_layout_passes=False),
    scratch_types=[
        pltpu.VMEM((R,), jnp.int32),
        pltpu.VMEM((R,), jnp.int32),
        pltpu.VMEM((R, D), jnp.float32),
        pltpu.VMEM((R, D), jnp.float32),
        pltpu.VMEM((R, D), jnp.float32),
        pltpu.VMEM((R, D), jnp.float32),
        pltpu.VMEM((L * L,), jnp.float32),
        pltpu.VMEM((L,), jnp.float32),
        pltpu.SemaphoreType.DMA,
        pltpu.SemaphoreType.DMA,
        pltpu.SemaphoreType.DMA,
        pltpu.SemaphoreType.DMA,
    ],
)
def _center_loss_sc(x_hbm, labels_hbm, centers_hbm, out_hbm,
                    idx0, idx1, xb0, xb1, cb0, cb1, trans_v, res_v,
                    sem_x0, sem_x1, sem_c0, sem_c1):
    cid = lax.axis_index("c")
    sid = lax.axis_index("s")
    wid = cid * NS + sid
    base = wid * RPW
    iota = lax.iota(jnp.int32, L)

    idxs = (idx0, idx1)
    xbs = (xb0, xb1)
    cbs = (cb0, cb1)
    sxs = (sem_x0, sem_x1)
    scs = (sem_c0, sem_c1)

    def issue(g, b):
        row0 = base + g * R
        pltpu.async_copy(x_hbm.at[pl.ds(row0, R)], xbs[b], sxs[b])
        pltpu.sync_copy(labels_hbm.at[pl.ds(row0, R)], idxs[b])
        pltpu.async_copy(centers_hbm.at[idxs[b]], cbs[b], scs[b])

    def wait(g, b):
        row0 = base + g * R
        pltpu.make_async_copy(x_hbm.at[pl.ds(row0, R)], xbs[b], sxs[b]).wait()
        pltpu.make_async_copy(centers_hbm.at[idxs[b]], cbs[b], scs[b]).wait()

    def compute(b, totv):
        x_buf = xbs[b]
        c_buf = cbs[b]
        def group_body(h, tv):
            def row_body(r16, _):
                r = h * L + r16
                accs = [jnp.zeros((L,), jnp.float32) for _ in range(4)]
                for j in range(DV):
                    d = x_buf[r, pl.ds(j * L, L)] - c_buf[r, pl.ds(j * L, L)]
                    accs[j % 4] = accs[j % 4] + d * d
                trans_v[pl.ds(r16 * L, L)] = (accs[0] + accs[1]) + (accs[2] + accs[3])
                return 0

            lax.fori_loop(0, L, row_body, 0)

            tsum = plsc.load_gather(trans_v, [iota * L])
            for k in range(1, L):
                tsum = tsum + plsc.load_gather(trans_v, [iota * L + k])
            tclip = jnp.minimum(jnp.maximum(tsum, 1e-12), 1e12)
            return tv + tclip

        return lax.fori_loop(0, NG, group_body, totv)

    issue(0, 0)

    def pair_body(t, totv):
        g0 = t * 2
        for b in range(2):
            gg = g0 + b

            @pl.when(gg + 1 < NCHUNK)
            def _():
                issue(gg + 1, 1 - b)

            wait(gg, b)
            totv = compute(b, totv)
        return totv

    totv = lax.fori_loop(0, NCHUNK // 2, pair_body, jnp.zeros((L,), jnp.float32))

    res_v[...] = totv
    pltpu.sync_copy(res_v, out_hbm.at[wid])


def _reduce_tc_body(p_ref, o_ref):
    o_ref[0, 0] = jnp.sum(p_ref[...]) * jnp.float32(1.0 / B)


_reduce_tc = pl.pallas_call(
    _reduce_tc_body,
    out_shape=jax.ShapeDtypeStruct((1, 1), jnp.float32),
    in_specs=[pl.BlockSpec(memory_space=pltpu.VMEM)],
    out_specs=pl.BlockSpec(memory_space=pltpu.SMEM),
)


def kernel(x, labels, centers):
    partials = _center_loss_sc(x, labels.astype(jnp.int32), centers)
    return _reduce_tc(partials)[0, 0]
